# Initial kernel scaffold; baseline (speedup 1.0000x reference)
import jax
import jax.numpy as jnp
from jax.experimental import pallas as pl


def kernel(wordid, emb):
    raise NotImplementedError("write your pallas kernel here")


# speedup vs baseline: 2.8486x; 2.8486x over previous
import functools

import jax
import jax.numpy as jnp
from jax.experimental import pallas as pl
from jax.experimental.pallas import tpu as pltpu

_VOCAB = 100000
_D = 128
_K = 65

_NB = 16
_BROWS = 6272
_SROWS = _BROWS // 128
_TOT_SROWS = _NB * _SROWS

_NEG = float("-inf")


def _body(wid_ref, wv_blk_ref, emb_blk_ref, out_s_ref, out_i_ref, scores_scr):
    i = pl.program_id(0)

    wv = wv_blk_ref[pl.ds(wid_ref[0] % 8, 1), :]

    scores = jax.lax.dot_general(
        emb_blk_ref[...].astype(jnp.bfloat16).astype(jnp.float32),
        wv.astype(jnp.bfloat16).astype(jnp.float32),
        dimension_numbers=(((1,), (1,)), ((), ())),
        preferred_element_type=jnp.float32,
    )
    s2 = scores.reshape(_SROWS, 128)

    r_io = jax.lax.broadcasted_iota(jnp.int32, (_SROWS, 128), 0)
    c_io = jax.lax.broadcasted_iota(jnp.int32, (_SROWS, 128), 1)
    gidx = i * _BROWS + r_io * 128 + c_io
    s2 = jnp.where(gidx < _VOCAB, s2, _NEG)
    scores_scr[pl.ds(i * _SROWS, _SROWS), :] = s2

    @pl.when(i == _NB - 1)
    def _():
        r2 = jax.lax.broadcasted_iota(jnp.int32, (_TOT_SROWS, 128), 0)
        c2 = jax.lax.broadcasted_iota(jnp.int32, (_TOT_SROWS, 128), 1)
        lin = r2 * 128 + c2
        k_io = jax.lax.broadcasted_iota(jnp.int32, (_K,), 0)
        big = jnp.int32(2**30)

        def step(k, _):
            s = scores_scr[...]
            m = jnp.max(s)
            hit = s == m
            idx = jnp.min(jnp.where(hit, lin, big))
            out_s_ref[...] = jnp.where(k_io == k, m, out_s_ref[...])
            out_i_ref[...] = jnp.where(k_io == k, idx, out_i_ref[...])
            scores_scr[...] = jnp.where(lin == idx, _NEG, s)
            return 0

        jax.lax.fori_loop(0, _K, step, 0)


@jax.jit
def kernel(wordid, emb):
    wid = wordid.astype(jnp.int32)
    grid_spec = pltpu.PrefetchScalarGridSpec(
        num_scalar_prefetch=1,
        grid=(_NB,),
        in_specs=[
            pl.BlockSpec((8, 128), lambda i, w: (w[0] // 8, 0)),
            pl.BlockSpec((_BROWS, 128), lambda i, w: (i, 0)),
        ],
        out_specs=[
            pl.BlockSpec((_K,), lambda i, w: (0,)),
            pl.BlockSpec((_K,), lambda i, w: (0,)),
        ],
        scratch_shapes=[pltpu.VMEM((_TOT_SROWS, 128), jnp.float32)],
    )
    scores, ids = pl.pallas_call(
        _body,
        grid_spec=grid_spec,
        out_shape=[
            jax.ShapeDtypeStruct((_K,), jnp.float32),
            jax.ShapeDtypeStruct((_K,), jnp.int32),
        ],
    )(wid, emb, emb)
    return scores, ids


# speedup vs baseline: 2.8961x; 1.0167x over previous
import jax
import jax.numpy as jnp
from jax.experimental import pallas as pl
from jax.experimental.pallas import tpu as pltpu

_VOCAB = 100000
_D = 128
_K = 65

_NB = 8
_SROWS = 104
_BROWS = _SROWS * 128
_TOT_SROWS = _NB * _SROWS
_NGRP = _TOT_SROWS // 8
_GPB = _SROWS // 8

_NEG = float("-inf")


def _body(wid_ref, wv_blk_ref, emb_blk_ref, out_s_ref, out_i_ref,
          scores_scr, gm_scr):
    i = pl.program_id(0)

    wv = wv_blk_ref[pl.ds(wid_ref[0] % 8, 1), :]

    scores = jax.lax.dot_general(
        emb_blk_ref[...].astype(jnp.bfloat16).astype(jnp.float32),
        wv.astype(jnp.bfloat16).astype(jnp.float32),
        dimension_numbers=(((1,), (1,)), ((), ())),
        preferred_element_type=jnp.float32,
    )
    s2 = scores.reshape(_SROWS, 128)

    r_io = jax.lax.broadcasted_iota(jnp.int32, (_SROWS, 128), 0)
    c_io = jax.lax.broadcasted_iota(jnp.int32, (_SROWS, 128), 1)
    gidx = i * _BROWS + r_io * 128 + c_io
    s2 = jnp.where(gidx < _VOCAB, s2, _NEG)
    scores_scr[pl.ds(i * _SROWS, _SROWS), :] = s2

    gm_blk = jnp.max(s2.reshape(_GPB, 8, 128), axis=1)
    gm_scr[pl.ds(i * _GPB, _GPB), :] = gm_blk

    @pl.when(i == _NB - 1)
    def _():
        lin_g = jax.lax.broadcasted_iota(jnp.int32, (_NGRP, 128), 0) * 128 + \
                jax.lax.broadcasted_iota(jnp.int32, (_NGRP, 128), 1)
        r8 = jax.lax.broadcasted_iota(jnp.int32, (8, 128), 0)
        c8 = jax.lax.broadcasted_iota(jnp.int32, (8, 128), 1)
        k_io = jax.lax.broadcasted_iota(jnp.int32, (_K,), 0)
        big = jnp.int32(2**30)

        def step(k, _):
            g = gm_scr[...]
            m = jnp.max(g)
            eg = jnp.min(jnp.where(g == m, lin_g, big))
            grp = eg // 128
            c = eg - grp * 128

            rows = scores_scr[pl.ds(grp * 8, 8), :]
            hit = (rows == m) & (c8 == c)
            r = jnp.min(jnp.where(hit, r8, big))
            gid = (grp * 8 + r) * 128 + c

            out_s_ref[...] = jnp.where(k_io == k, m, out_s_ref[...])
            out_i_ref[...] = jnp.where(k_io == k, gid, out_i_ref[...])

            rows = jnp.where((r8 == r) & (c8 == c), _NEG, rows)
            scores_scr[pl.ds(grp * 8, 8), :] = rows
            gm_scr[pl.ds(grp, 1), :] = jnp.max(rows, axis=0, keepdims=True)
            return 0

        jax.lax.fori_loop(0, _K, step, 0)


@jax.jit
def kernel(wordid, emb):
    wid = wordid.astype(jnp.int32)
    grid_spec = pltpu.PrefetchScalarGridSpec(
        num_scalar_prefetch=1,
        grid=(_NB,),
        in_specs=[
            pl.BlockSpec((8, 128), lambda i, w: (w[0] // 8, 0)),
            pl.BlockSpec((_BROWS, 128), lambda i, w: (i, 0)),
        ],
        out_specs=[
            pl.BlockSpec((_K,), lambda i, w: (0,)),
            pl.BlockSpec((_K,), lambda i, w: (0,)),
        ],
        scratch_shapes=[
            pltpu.VMEM((_TOT_SROWS, 128), jnp.float32),
            pltpu.VMEM((_NGRP, 128), jnp.float32),
        ],
    )
    scores, ids = pl.pallas_call(
        _body,
        grid_spec=grid_spec,
        out_shape=[
            jax.ShapeDtypeStruct((_K,), jnp.float32),
            jax.ShapeDtypeStruct((_K,), jnp.int32),
        ],
    )(wid, emb, emb)
    return scores, ids


# speedup vs baseline: 2.9623x; 1.0228x over previous
import jax
import jax.numpy as jnp
from jax.experimental import pallas as pl
from jax.experimental.pallas import tpu as pltpu

_VOCAB = 100000
_D = 128
_K = 65

_NB = 4
_SROWS = 200
_BROWS = _SROWS * 128
_TOT_SROWS = _NB * _SROWS
_NGRP = _TOT_SROWS // 8
_GPB = _SROWS // 8

_NEG = float("-inf")


def _body(wid_ref, wv_blk_ref, emb_blk_ref, out_s_ref, out_i_ref,
          scores_scr, gm_scr):
    i = pl.program_id(0)

    wv = wv_blk_ref[pl.ds(wid_ref[0] % 8, 1), :]

    scores = jax.lax.dot_general(
        emb_blk_ref[...].astype(jnp.bfloat16).astype(jnp.float32),
        wv.astype(jnp.bfloat16).astype(jnp.float32),
        dimension_numbers=(((1,), (1,)), ((), ())),
        preferred_element_type=jnp.float32,
    )
    s2 = scores.reshape(_SROWS, 128)

    r_io = jax.lax.broadcasted_iota(jnp.int32, (_SROWS, 128), 0)
    c_io = jax.lax.broadcasted_iota(jnp.int32, (_SROWS, 128), 1)
    gidx = i * _BROWS + r_io * 128 + c_io
    s2 = jnp.where(gidx < _VOCAB, s2, _NEG)
    scores_scr[pl.ds(i * _SROWS, _SROWS), :] = s2

    gm_blk = jnp.max(s2.reshape(_GPB, 8, 128), axis=1)
    gm_scr[pl.ds(i * _GPB, _GPB), :] = gm_blk

    @pl.when(i == _NB - 1)
    def _():
        lin_g = jax.lax.broadcasted_iota(jnp.int32, (_NGRP, 128), 0) * 128 + \
                jax.lax.broadcasted_iota(jnp.int32, (_NGRP, 128), 1)
        r8 = jax.lax.broadcasted_iota(jnp.int32, (8, 128), 0)
        c8 = jax.lax.broadcasted_iota(jnp.int32, (8, 128), 1)
        k_io = jax.lax.broadcasted_iota(jnp.int32, (_K,), 0)
        big = jnp.int32(2**30)

        def step(k, _):
            g = gm_scr[...]
            m = jnp.max(g)
            eg = jnp.min(jnp.where(g == m, lin_g, big))
            grp = eg // 128
            c = eg - grp * 128

            rows = scores_scr[pl.ds(grp * 8, 8), :]
            hit = (rows == m) & (c8 == c)
            r = jnp.min(jnp.where(hit, r8, big))
            gid = (grp * 8 + r) * 128 + c

            out_s_ref[...] = jnp.where(k_io == k, m, out_s_ref[...])
            out_i_ref[...] = jnp.where(k_io == k, gid, out_i_ref[...])

            rows = jnp.where((r8 == r) & (c8 == c), _NEG, rows)
            scores_scr[pl.ds(grp * 8, 8), :] = rows
            gm_scr[pl.ds(grp, 1), :] = jnp.max(rows, axis=0, keepdims=True)
            return 0

        jax.lax.fori_loop(0, _K, step, 0)


@jax.jit
def kernel(wordid, emb):
    wid = wordid.astype(jnp.int32)
    grid_spec = pltpu.PrefetchScalarGridSpec(
        num_scalar_prefetch=1,
        grid=(_NB,),
        in_specs=[
            pl.BlockSpec((8, 128), lambda i, w: (w[0] // 8, 0)),
            pl.BlockSpec((_BROWS, 128), lambda i, w: (i, 0)),
        ],
        out_specs=[
            pl.BlockSpec((_K,), lambda i, w: (0,)),
            pl.BlockSpec((_K,), lambda i, w: (0,)),
        ],
        scratch_shapes=[
            pltpu.VMEM((_TOT_SROWS, 128), jnp.float32),
            pltpu.VMEM((_NGRP, 128), jnp.float32),
        ],
    )
    scores, ids = pl.pallas_call(
        _body,
        grid_spec=grid_spec,
        out_shape=[
            jax.ShapeDtypeStruct((_K,), jnp.float32),
            jax.ShapeDtypeStruct((_K,), jnp.int32),
        ],
    )(wid, emb, emb)
    return scores, ids
